# baseline (speedup 1.0000x reference)
import functools

import jax
import jax.numpy as jnp
from jax import lax
from jax.experimental import pallas as pl
from jax.experimental.pallas import tpu as pltpu
from jax.experimental.pallas import tpu_sc as plsc

V = 23236
D = 1728
BSZ = 64
SEQ = 512
NC = 2
NS = 16
NW = NC * NS
F_PER = D // NW
NPAIR = F_PER // 2
NBLK = BSZ // 2
NOB = 4


@jax.jit
def _sc_lookup(ids_u16, table_t):
    mesh = plsc.VectorSubcoreMesh(
        core_axis_name="c", subcore_axis_name="s", num_cores=NC, num_subcores=NS
    )

    @functools.partial(
        pl.kernel,
        out_type=jax.ShapeDtypeStruct((BSZ, D, SEQ), jnp.float32),
        mesh=mesh,
        scratch_types=[
            pltpu.VMEM((BSZ * SEQ // 2,), jnp.int32),
            [pltpu.VMEM((2, V), jnp.float32) for _ in range(2)],
            [pltpu.VMEM((2, 2, SEQ), jnp.float32) for _ in range(NOB)],
            [pltpu.SemaphoreType.DMA for _ in range(2)],
            [pltpu.SemaphoreType.DMA for _ in range(NOB)],
        ],
        compiler_params=pltpu.CompilerParams(
            use_tc_tiling_on_sc=True, needs_layout_passes=False
        ),
    )
    def k(ids_hbm, table_hbm, out_hbm, ids_v, quads, obufs, rsem, osem):
        wid = lax.axis_index("s") * NC + lax.axis_index("c")
        f0 = wid * F_PER
        row0 = jnp.zeros((16,), jnp.int32)
        row1 = jnp.ones((16,), jnp.int32)
        mask16 = jnp.full((16,), 0xFFFF, jnp.int32)
        sh16 = jnp.full((16,), 16, jnp.int32)
        pltpu.sync_copy(ids_hbm, ids_v)

        def process_pair(p, r):
            f = f0 + p * 2
            pltpu.make_async_copy(
                table_hbm.at[pl.ds(f0, 2)], quads[r], rsem[r]
            ).wait()

            @pl.when(p + 1 < NPAIR)
            def _():
                pltpu.async_copy(
                    table_hbm.at[pl.ds(f + 2, 2)], quads[1 - r], rsem[1 - r]
                )

            def blk(q, _):
                for sub in range(NOB):
                    bb = q * NOB + sub

                    @pl.when((p > 0) | (q > 0))
                    def _():
                        pltpu.make_async_copy(
                            obufs[sub],
                            out_hbm.at[pl.ds(0, 2), pl.ds(0, 2)],
                            osem[sub],
                        ).wait()

                    base = bb * SEQ

                    @plsc.parallel_loop(0, 2 * SEQ // 32, unroll=8)
                    def _(u):
                        packed = ids_v[pl.ds(base + u * 16, 16)]
                        idx_lo = packed & mask16
                        idx_hi = lax.shift_right_logical(packed, sh16)
                        b_loc = u >> 4
                        s_off = (u & 15) * 32
                        for row, rv in ((row0, 0), (row1, 1)):
                            obufs[sub][b_loc, rv, pl.ds(s_off, 16)] = (
                                plsc.load_gather(quads[r], [row, idx_lo])
                            )
                            obufs[sub][b_loc, rv, pl.ds(s_off + 16, 16)] = (
                                plsc.load_gather(quads[r], [row, idx_hi])
                            )

                    pltpu.async_copy(
                        obufs[sub],
                        out_hbm.at[pl.ds(bb * 2, 2), pl.ds(f, 2)],
                        osem[sub],
                    )
                return _

            lax.fori_loop(0, NBLK // NOB, blk, None)

        pltpu.async_copy(table_hbm.at[pl.ds(f0, 2)], quads[0], rsem[0])

        def pair2(i, _):
            process_pair(i * 2, 0)
            process_pair(i * 2 + 1, 1)
            return _

        lax.fori_loop(0, NPAIR // 2, pair2, None)
        process_pair(NPAIR - 1, 0)

        for sub in range(NOB):
            pltpu.make_async_copy(
                obufs[sub], out_hbm.at[pl.ds(0, 2), pl.ds(0, 2)], osem[sub]
            ).wait()

    return k(ids_u16, table_t)


def kernel(input_ids, font_weights):
    ids = input_ids.astype(jnp.int32).reshape(-1, 2, 16)
    ids_packed = ids[:, 0, :] | (ids[:, 1, :] << 16)
    table_t = font_weights.T
    out_t = _sc_lookup(ids_packed.reshape(-1), table_t)
    return jnp.transpose(out_t, (0, 2, 1))


# speedup vs baseline: 1.0154x; 1.0154x over previous
import functools

import jax
import jax.numpy as jnp
from jax import lax
from jax.experimental import pallas as pl
from jax.experimental.pallas import tpu as pltpu
from jax.experimental.pallas import tpu_sc as plsc

V = 23236
D = 1728
BSZ = 64
SEQ = 512
NC = 2
NS = 16
NW = NC * NS
F_PER = D // NW
NPAIR = F_PER // 2
NBLK = BSZ // 2


@jax.jit
def _sc_lookup(ids, table_t):
    mesh = plsc.VectorSubcoreMesh(
        core_axis_name="c", subcore_axis_name="s", num_cores=NC, num_subcores=NS
    )

    @functools.partial(
        pl.kernel,
        out_type=jax.ShapeDtypeStruct((BSZ, D, SEQ), jnp.float32),
        mesh=mesh,
        scratch_types=[
            pltpu.VMEM((BSZ * SEQ,), jnp.int32),
            [pltpu.VMEM((2, V), jnp.float32) for _ in range(2)],
            [pltpu.VMEM((2, 2, SEQ), jnp.float32) for _ in range(2)],
            [pltpu.SemaphoreType.DMA for _ in range(2)],
            [pltpu.SemaphoreType.DMA for _ in range(2)],
        ],
        compiler_params=pltpu.CompilerParams(
            use_tc_tiling_on_sc=True, needs_layout_passes=False
        ),
    )
    def k(ids_hbm, table_hbm, out_hbm, ids_v, quads, obufs, rsem, osem):
        wid = lax.axis_index("s") * NC + lax.axis_index("c")
        f0 = wid * F_PER
        row0 = jnp.zeros((16,), jnp.int32)
        row1 = jnp.ones((16,), jnp.int32)
        pltpu.sync_copy(ids_hbm, ids_v)

        def process_pair(p, r):
            f = f0 + p * 2
            pltpu.make_async_copy(
                table_hbm.at[pl.ds(f0, 2)], quads[r], rsem[r]
            ).wait()

            @pl.when(p + 1 < NPAIR)
            def _():
                pltpu.async_copy(
                    table_hbm.at[pl.ds(f + 2, 2)], quads[1 - r], rsem[1 - r]
                )

            def blk(q, _):
                for sub in range(2):
                    bb = q * 2 + sub

                    @pl.when((p > 0) | (q > 0))
                    def _():
                        pltpu.make_async_copy(
                            obufs[sub],
                            out_hbm.at[pl.ds(0, 2), pl.ds(0, 2)],
                            osem[sub],
                        ).wait()

                    base = bb * (2 * SEQ)

                    @plsc.parallel_loop(0, 2 * SEQ // 16, unroll=16)
                    def _(u):
                        idx = ids_v[pl.ds(base + u * 16, 16)]
                        b_loc = u >> 5
                        s_off = (u & 31) * 16
                        obufs[sub][b_loc, 0, pl.ds(s_off, 16)] = (
                            plsc.load_gather(quads[r], [row0, idx])
                        )
                        obufs[sub][b_loc, 1, pl.ds(s_off, 16)] = (
                            plsc.load_gather(quads[r], [row1, idx])
                        )

                    pltpu.async_copy(
                        obufs[sub],
                        out_hbm.at[pl.ds(bb * 2, 2), pl.ds(f, 2)],
                        osem[sub],
                    )
                return _

            lax.fori_loop(0, NBLK // 2, blk, None)

        pltpu.async_copy(table_hbm.at[pl.ds(f0, 2)], quads[0], rsem[0])

        def pair2(i, _):
            process_pair(i * 2, 0)
            process_pair(i * 2 + 1, 1)
            return _

        lax.fori_loop(0, NPAIR // 2, pair2, None)
        process_pair(NPAIR - 1, 0)

        for sub in range(2):
            pltpu.make_async_copy(
                obufs[sub], out_hbm.at[pl.ds(0, 2), pl.ds(0, 2)], osem[sub]
            ).wait()

    return k(ids, table_t)


def kernel(input_ids, font_weights):
    ids = input_ids.reshape(-1).astype(jnp.int32)
    table_t = font_weights.T
    out_t = _sc_lookup(ids, table_t)
    return jnp.transpose(out_t, (0, 2, 1))


# speedup vs baseline: 1.0305x; 1.0149x over previous
import functools

import jax
import jax.numpy as jnp
from jax import lax
from jax.experimental import pallas as pl
from jax.experimental.pallas import tpu as pltpu
from jax.experimental.pallas import tpu_sc as plsc

V = 23236
D = 1728
BSZ = 64
SEQ = 512
NC = 2
NS = 16
NW = NC * NS
F_PER = D // NW
NPAIR = F_PER // 2
NBLK = BSZ // 2


@jax.jit
def _sc_lookup(ids, table_t):
    mesh = plsc.VectorSubcoreMesh(
        core_axis_name="c", subcore_axis_name="s", num_cores=NC, num_subcores=NS
    )

    @functools.partial(
        pl.kernel,
        out_type=jax.ShapeDtypeStruct((BSZ, D, SEQ), jnp.float32),
        mesh=mesh,
        scratch_types=[
            pltpu.VMEM((BSZ * SEQ,), jnp.int32),
            [pltpu.VMEM((2, V), jnp.float32) for _ in range(2)],
            [pltpu.VMEM((2, 2, SEQ), jnp.float32) for _ in range(2)],
            [pltpu.SemaphoreType.DMA for _ in range(2)],
            [pltpu.SemaphoreType.DMA for _ in range(2)],
        ],
        compiler_params=pltpu.CompilerParams(
            use_tc_tiling_on_sc=True, needs_layout_passes=False
        ),
    )
    def k(ids_hbm, table_hbm, out_hbm, ids_v, quads, obufs, rsem, osem):
        wid = lax.axis_index("s") * NC + lax.axis_index("c")
        f0 = wid * F_PER
        row0 = jnp.zeros((16,), jnp.int32)
        row1 = jnp.ones((16,), jnp.int32)
        pltpu.sync_copy(ids_hbm, ids_v)

        def process_pair(p, r):
            f = f0 + p * 2
            pltpu.make_async_copy(
                table_hbm.at[pl.ds(f0, 2)], quads[r], rsem[r]
            ).wait()

            @pl.when(p + 1 < NPAIR)
            def _():
                pltpu.async_copy(
                    table_hbm.at[pl.ds(f + 2, 2)], quads[1 - r], rsem[1 - r]
                )

            def blk(q, _):
                for sub in range(2):
                    bb = q * 2 + sub

                    @pl.when((p > 0) | (q > 0))
                    def _():
                        pltpu.make_async_copy(
                            obufs[sub],
                            out_hbm.at[pl.ds(0, 2), pl.ds(0, 2)],
                            osem[sub],
                        ).wait()

                    base = bb * (2 * SEQ)

                    @plsc.parallel_loop(0, 2 * SEQ // 16, unroll=8)
                    def _(u):
                        idx = ids_v[pl.ds(base + u * 16, 16)]
                        b_loc = u >> 5
                        s_off = (u & 31) * 16
                        obufs[sub][b_loc, 0, pl.ds(s_off, 16)] = (
                            plsc.load_gather(quads[r], [row0, idx])
                        )
                        obufs[sub][b_loc, 1, pl.ds(s_off, 16)] = (
                            plsc.load_gather(quads[r], [row1, idx])
                        )

                    pltpu.async_copy(
                        obufs[sub],
                        out_hbm.at[pl.ds(bb * 2, 2), pl.ds(f, 2)],
                        osem[sub],
                    )
                return _

            lax.fori_loop(0, NBLK // 2, blk, None)

        pltpu.async_copy(table_hbm.at[pl.ds(f0, 2)], quads[0], rsem[0])

        def pair2(i, _):
            process_pair(i * 2, 0)
            process_pair(i * 2 + 1, 1)
            return _

        lax.fori_loop(0, NPAIR // 2, pair2, None)
        process_pair(NPAIR - 1, 0)

        for sub in range(2):
            pltpu.make_async_copy(
                obufs[sub], out_hbm.at[pl.ds(0, 2), pl.ds(0, 2)], osem[sub]
            ).wait()

    return k(ids, table_t)


def kernel(input_ids, font_weights):
    ids = input_ids.reshape(-1).astype(jnp.int32)
    table_t = font_weights.T
    out_t = _sc_lookup(ids, table_t)
    return jnp.transpose(out_t, (0, 2, 1))


# speedup vs baseline: 1.0325x; 1.0019x over previous
import functools

import jax
import jax.numpy as jnp
from jax import lax
from jax.experimental import pallas as pl
from jax.experimental.pallas import tpu as pltpu
from jax.experimental.pallas import tpu_sc as plsc

V = 23236
D = 1728
BSZ = 64
SEQ = 512
NC = 2
NS = 16
NW = NC * NS
F_PER = D // NW
NPAIR = F_PER // 2
NBLK = BSZ // 2


@jax.jit
def _sc_lookup(ids, table_t):
    mesh = plsc.VectorSubcoreMesh(
        core_axis_name="c", subcore_axis_name="s", num_cores=NC, num_subcores=NS
    )

    @functools.partial(
        pl.kernel,
        out_type=jax.ShapeDtypeStruct((BSZ, D, SEQ), jnp.float32),
        mesh=mesh,
        scratch_types=[
            pltpu.VMEM((BSZ * SEQ,), jnp.int32),
            [pltpu.VMEM((2, V), jnp.float32) for _ in range(2)],
            [pltpu.VMEM((2, 2, SEQ), jnp.float32) for _ in range(2)],
            [pltpu.SemaphoreType.DMA for _ in range(2)],
            [pltpu.SemaphoreType.DMA for _ in range(2)],
        ],
        compiler_params=pltpu.CompilerParams(
            use_tc_tiling_on_sc=True, needs_layout_passes=False
        ),
    )
    def k(ids_hbm, table_hbm, out_hbm, ids_v, quads, obufs, rsem, osem):
        wid = lax.axis_index("s") * NC + lax.axis_index("c")
        f0 = wid * F_PER
        row0 = jnp.zeros((16,), jnp.int32)
        row1 = jnp.ones((16,), jnp.int32)
        pltpu.async_copy(table_hbm.at[pl.ds(f0, 2)], quads[0], rsem[0])
        pltpu.sync_copy(ids_hbm, ids_v)

        def process_pair(p, r):
            f = f0 + p * 2
            pltpu.make_async_copy(
                table_hbm.at[pl.ds(f0, 2)], quads[r], rsem[r]
            ).wait()

            @pl.when(p + 1 < NPAIR)
            def _():
                pltpu.async_copy(
                    table_hbm.at[pl.ds(f + 2, 2)], quads[1 - r], rsem[1 - r]
                )

            def blk(q, _):
                for sub in range(2):
                    bb = q * 2 + sub

                    @pl.when((p > 0) | (q > 0))
                    def _():
                        pltpu.make_async_copy(
                            obufs[sub],
                            out_hbm.at[pl.ds(0, 2), pl.ds(0, 2)],
                            osem[sub],
                        ).wait()

                    base = bb * (2 * SEQ)

                    @plsc.parallel_loop(0, 2 * SEQ // 16, unroll=8)
                    def _(u):
                        idx = ids_v[pl.ds(base + u * 16, 16)]
                        b_loc = u >> 5
                        s_off = (u & 31) * 16
                        obufs[sub][b_loc, 0, pl.ds(s_off, 16)] = (
                            plsc.load_gather(quads[r], [row0, idx])
                        )
                        obufs[sub][b_loc, 1, pl.ds(s_off, 16)] = (
                            plsc.load_gather(quads[r], [row1, idx])
                        )

                    pltpu.async_copy(
                        obufs[sub],
                        out_hbm.at[pl.ds(bb * 2, 2), pl.ds(f, 2)],
                        osem[sub],
                    )
                return _

            lax.fori_loop(0, NBLK // 2, blk, None)

        def pair2(i, _):
            process_pair(i * 2, 0)
            process_pair(i * 2 + 1, 1)
            return _

        lax.fori_loop(0, NPAIR // 2, pair2, None)
        process_pair(NPAIR - 1, 0)

        for sub in range(2):
            pltpu.make_async_copy(
                obufs[sub], out_hbm.at[pl.ds(0, 2), pl.ds(0, 2)], osem[sub]
            ).wait()

    return k(ids, table_t)


def kernel(input_ids, font_weights):
    ids = input_ids.reshape(-1).astype(jnp.int32)
    table_t = font_weights.T
    out_t = _sc_lookup(ids, table_t)
    return jnp.transpose(out_t, (0, 2, 1))
